# Initial kernel scaffold; baseline (speedup 1.0000x reference)
import jax
import jax.numpy as jnp
from jax.experimental import pallas as pl


def kernel(xl, xh, te, localadj, params):
    raise NotImplementedError("write your pallas kernel here")


# speedup vs baseline: 8.0332x; 8.0332x over previous
import math

import jax
import jax.numpy as jnp
from jax import lax
from jax.experimental import pallas as pl

B, T, N, H, D = 2, 12, 512, 8, 16
F = H * D
LA = 8
NSAMP = int(2 * math.log2(N))
NB_BLK = 128
NBLKS = N // NB_BLK
_INTERPRET = False


def _lnf(x, eps=1e-5):
    m = jnp.mean(x, 0, keepdims=True)
    v = jnp.mean((x - m) ** 2, 0, keepdims=True)
    return (x - m) / jnp.sqrt(v + eps)


def _b16(x):
    return x.astype(jnp.bfloat16)


def _b16f(x):
    return x.astype(jnp.bfloat16).astype(jnp.float32)


def _mm(w, x):
    return jnp.dot(_b16(w), _b16(x), preferred_element_type=jnp.float32)


def _k1_body(xl_ref, te_ref, xh_ref,
             qW, qb, kW, kb, vW, vb, oW, ob, f1W, f1b, f2W, f2b,
             cW0, cW1, cb,
             xl1_ref, xh1_ref):
    nb = xl_ref.shape[-1]
    tn = T * nb
    x3 = xl_ref[0] + te_ref[0]
    xm = x3.reshape(F, tn)
    q = (_mm(qW[...], xm) + qb[...]).reshape(H, D, T, nb)
    k = (_mm(kW[...], xm) + kb[...]).reshape(H, D, T, nb)
    v = (_mm(vW[...], xm) + vb[...]).reshape(H, D, T, nb)
    ti = lax.broadcasted_iota(jnp.int32, (T, T, 1), 0)
    si = lax.broadcasted_iota(jnp.int32, (T, T, 1), 1)
    causal = si <= ti
    scale = 1.0 / math.sqrt(D)
    heads = []
    for h in range(H):
        qh, kh, vh = _b16f(q[h]), _b16f(k[h]), _b16f(v[h])
        att = (qh[:, :, None, :] * kh[:, None, :, :]).sum(0) * scale
        att = jnp.where(causal, att, -32767.0)
        mx = att.max(1, keepdims=True)
        e = jnp.exp(att - mx)
        att = _b16f(e / e.sum(1, keepdims=True))
        heads.append((att[None] * vh[:, None, :, :]).sum(2))
    val = jnp.concatenate(heads, axis=0).reshape(F, tn)
    o = _mm(oW[...], val) + ob[...]
    y = _lnf(o + xm)
    h1 = jax.nn.relu(_mm(f1W[...], y) + f1b[...])
    h2 = _mm(f2W[...], h1) + f2b[...]
    xl1_ref[...] = _lnf(h2 + y).reshape(1, F, T, nb)

    xhm = xh_ref[0].reshape(F, tn)
    y1 = (_mm(cW1[...], xhm) + cb[...]).reshape(F, T, nb)
    y0 = _mm(cW0[...], xhm).reshape(F, T, nb)
    y0s = jnp.concatenate([jnp.zeros((F, 1, nb), jnp.float32), y0[:, : T - 1, :]], axis=1)
    xh1_ref[...] = jax.nn.relu(y1 + y0s).reshape(1, F, T, nb)


def _k2_body(xl1_ref, xh1_ref, adjT_ref, svecT_ref, tvecT_ref, sval, tval,
             lqW, lqb, lkW, lkb, lvW, lvb,
             hqW, hqb, hkW, hkb, hvW, hvb,
             ql_ref, kl_ref, vl_ref, qh_ref, kh_ref, vh_ref, oh_ref):
    nb = xl1_ref.shape[-1]
    tn = T * nb
    a1 = svecT_ref[...] * sval[...]
    a2 = tvecT_ref[...] * tval[...]
    ci = lax.broadcasted_iota(jnp.int32, (N, nb), 0)
    for j in range(LA):
        oh_ref[j] = (ci == adjT_ref[j, :][None, :]).astype(jnp.bfloat16)

    for x_ref, qW, qb, kW, kb, vW, vb, q_ref, k_ref, v_ref in (
        (xl1_ref, lqW, lqb, lkW, lkb, lvW, lvb, ql_ref, kl_ref, vl_ref),
        (xh1_ref, hqW, hqb, hkW, hkb, hvW, hvb, qh_ref, kh_ref, vh_ref),
    ):
        xm = ((x_ref[0] + a1[:, None, :]) + a2[:, None, :]).reshape(F, tn)
        q_ref[...] = (_mm(qW[...], xm) + qb[...]).reshape(1, F, T, nb)
        k_ref[...] = (_mm(kW[...], xm) + kb[...]).reshape(1, F, T, nb)
        v_ref[...] = (_mm(vW[...], xm) + vb[...]).reshape(1, F, T, nb)


def _sparse_branch(q, k, v, oh_ref, pW, pb):
    qbf = _b16f(q)
    kbf = _b16f(k)
    vbf = _b16f(v)
    scale = 1.0 / math.sqrt(D)

    m = jnp.zeros((T, N), jnp.float32)
    for j in range(LA):
        ksj = jnp.dot(k.reshape(D * T, N), oh_ref[j].astype(jnp.float32),
                      preferred_element_type=jnp.float32,
                      precision=lax.Precision.HIGHEST).reshape(D, T, N)
        qks = (q * ksj).sum(0)
        m = m + _b16f(qks) * _b16f(pW[0, j])
    m = m + pb

    lane = lax.broadcasted_iota(jnp.int32, (T, N), 1)
    mm_ = m
    ohs = []
    for _ in range(NSAMP):
        mx = mm_.max(1, keepdims=True)
        idx = jnp.min(jnp.where(mm_ == mx, lane, N), 1, keepdims=True)
        sel = lane == idx
        ohs.append(sel)
        mm_ = jnp.where(sel, -1e30, mm_)

    best = None
    bidx = jnp.zeros((T, N), jnp.int32)
    vfs = []
    for i, sel in enumerate(ohs):
        self_f = sel.astype(jnp.float32)
        qred = (qbf * self_f[None]).sum(2)
        qk = (qred[:, :, None] * kbf).sum(0) * scale
        mx = qk.max(1, keepdims=True)
        e = jnp.exp(qk - mx)
        attn = e / e.sum(1, keepdims=True)
        vfs.append((_b16f(attn)[None] * vbf).sum(2))
        if best is None:
            best = attn
        else:
            gt = attn > best
            best = jnp.where(gt, attn, best)
            bidx = jnp.where(gt, i, bidx)

    acc = jnp.zeros((D, T, N), jnp.float32)
    for i, vf in enumerate(vfs):
        self_f = (bidx == i).astype(jnp.float32)
        acc = acc + vf[:, :, None] * self_f[None]
    return acc


def _k3_body(ql_ref, kl_ref, vl_ref, qh_ref, kh_ref, vh_ref,
             oh_ref, pWl, pbl, pWh, pbh, vall_ref, valh_ref):
    vall_ref[...] = _sparse_branch(
        ql_ref[0, 0], kl_ref[0, 0], vl_ref[0, 0], oh_ref, pWl, pbl[0, 0]
    )[None, None]
    valh_ref[...] = _sparse_branch(
        qh_ref[0, 0], kh_ref[0, 0], vh_ref[0, 0], oh_ref, pWh, pbh[0, 0]
    )[None, None]


def _k4_body(vall_ref, valh_ref, xl1_ref, xh1_ref,
             loW, lob, llng, llnb, lf1W, lf1b, lf2W, lf2b,
             hoW, hob, hlng, hlnb, hf1W, hf1b, hf2W, hf2b,
             outl_ref, outh_ref):
    nb = vall_ref.shape[-1]
    tn = T * nb
    for val_ref, x1_ref, oW, ob, lng, lnb, f1W, f1b, f2W, f2b, out_ref in (
        (vall_ref, xl1_ref, loW, lob, llng, llnb, lf1W, lf1b, lf2W, lf2b, outl_ref),
        (valh_ref, xh1_ref, hoW, hob, hlng, hlnb, hf1W, hf1b, hf2W, hf2b, outh_ref),
    ):
        vm = val_ref[0].reshape(F, tn)
        o = _mm(oW[...], vm) + ob[...]
        y = _lnf(o) * lng[...] + lnb[...]
        h1 = jax.nn.relu(_mm(f1W[...], y) + f1b[...])
        h2 = _mm(f2W[...], h1) + f2b[...]
        s = _lnf(h2 + y) + x1_ref[0].reshape(F, tn)
        out_ref[...] = s.reshape(1, F, T, nb)


def _full(shape):
    nd = len(shape)
    return pl.BlockSpec(shape, lambda *_: (0,) * nd)


def kernel(xl, xh, te, localadj, params):
    f32 = jnp.float32
    tem, spl, sph = params['tem'], params['spa_l'], params['spa_h']

    xl_f = jnp.transpose(xl, (0, 3, 1, 2))
    xh_f = jnp.transpose(xh, (0, 3, 1, 2))
    te_f = jnp.transpose(te, (0, 3, 1, 2))
    adjT = localadj.T
    svecT = params['spa_vec'].T
    tvecT = params['tem_vec'].T
    sval = params['spa_val'][:, None]
    tval = params['tem_val'][:, None]

    def col(p, names):
        out = []
        for nm in names:
            w = p[nm]
            out.append(w[:, None] if w.ndim == 1 else w)
        return out

    tem_args = col(tem, ['qW', 'qb', 'kW', 'kb', 'vW', 'vb', 'oW', 'ob',
                         'f1W', 'f1b', 'f2W', 'f2b'])
    cW0 = params['convW'][:, :, 0, 0]
    cW1 = params['convW'][:, :, 0, 1]
    cb = params['convb'][:, None]

    dblk = pl.BlockSpec((1, F, T, NB_BLK), lambda b, n: (b, 0, 0, n))
    dshape = jax.ShapeDtypeStruct((B, F, T, N), f32)

    xl1_f, xh1_f = pl.pallas_call(
        _k1_body,
        grid=(B, NBLKS),
        in_specs=[dblk, dblk, dblk] + [_full(a.shape) for a in tem_args]
                 + [_full(cW0.shape), _full(cW1.shape), _full(cb.shape)],
        out_specs=[dblk, dblk],
        out_shape=[dshape, dshape],
        interpret=_INTERPRET,
    )(xl_f, te_f, xh_f, *tem_args, cW0, cW1, cb)

    dblk2 = pl.BlockSpec((1, F, T, NB_BLK), lambda n, b: (b, 0, 0, n))
    qkv_args_l = col(spl, ['qW', 'qb', 'kW', 'kb', 'vW', 'vb'])
    qkv_args_h = col(sph, ['qW', 'qb', 'kW', 'kb', 'vW', 'vb'])
    k2_out = pl.pallas_call(
        _k2_body,
        grid=(NBLKS, B),
        in_specs=[dblk2, dblk2,
                  pl.BlockSpec((LA, NB_BLK), lambda n, b: (0, n)),
                  pl.BlockSpec((F, NB_BLK), lambda n, b: (0, n)),
                  pl.BlockSpec((F, NB_BLK), lambda n, b: (0, n)),
                  _full(sval.shape), _full(tval.shape)]
                 + [_full(a.shape) for a in qkv_args_l]
                 + [_full(a.shape) for a in qkv_args_h],
        out_specs=[dblk2] * 6
                  + [pl.BlockSpec((LA, N, NB_BLK), lambda n, b: (0, 0, n))],
        out_shape=[dshape] * 6
                  + [jax.ShapeDtypeStruct((LA, N, N), jnp.bfloat16)],
        interpret=_INTERPRET,
    )(xl1_f, xh1_f, adjT, svecT, tvecT, sval, tval, *qkv_args_l, *qkv_args_h)
    ql, kl, vl, qh, kh, vh, OH = k2_out

    h5 = lambda a: a.reshape(B, H, D, T, N)
    hblk = pl.BlockSpec((1, 1, D, T, N), lambda b, h: (b, h, 0, 0, 0))
    hshape = jax.ShapeDtypeStruct((B, H, D, T, N), f32)
    pbl = spl['projb'].reshape(1, 1)
    pbh = sph['projb'].reshape(1, 1)
    vall5, valh5 = pl.pallas_call(
        _k3_body,
        grid=(B, H),
        in_specs=[hblk] * 6 + [_full((LA, N, N)),
                               _full(spl['projW'].shape), _full((1, 1)),
                               _full(sph['projW'].shape), _full((1, 1))],
        out_specs=[hblk, hblk],
        out_shape=[hshape, hshape],
        interpret=_INTERPRET,
    )(h5(ql), h5(kl), h5(vl), h5(qh), h5(kh), h5(vh), OH,
      spl['projW'], pbl, sph['projW'], pbh)

    out_args_l = col(spl, ['oW', 'ob', 'lng', 'lnb', 'f1W', 'f1b', 'f2W', 'f2b'])
    out_args_h = col(sph, ['oW', 'ob', 'lng', 'lnb', 'f1W', 'f1b', 'f2W', 'f2b'])
    outl_f, outh_f = pl.pallas_call(
        _k4_body,
        grid=(B, NBLKS),
        in_specs=[dblk] * 4 + [_full(a.shape) for a in out_args_l]
                 + [_full(a.shape) for a in out_args_h],
        out_specs=[dblk, dblk],
        out_shape=[dshape, dshape],
        interpret=_INTERPRET,
    )(vall5.reshape(B, F, T, N), valh5.reshape(B, F, T, N), xl1_f, xh1_f,
      *out_args_l, *out_args_h)

    outl = jnp.transpose(outl_f, (0, 2, 3, 1))
    outh = jnp.transpose(outh_f, (0, 2, 3, 1))
    return (outl, outh)


# speedup vs baseline: 8.4274x; 1.0491x over previous
import math

import jax
import jax.numpy as jnp
from jax import lax
from jax.experimental import pallas as pl

B, T, N, H, D = 2, 12, 512, 8, 16
F = H * D
LA = 8
NSAMP = int(2 * math.log2(N))
NB_BLK = 128
NBLKS = N // NB_BLK
_INTERPRET = False


def _lnf(x, eps=1e-5):
    m = jnp.mean(x, 0, keepdims=True)
    v = jnp.mean((x - m) ** 2, 0, keepdims=True)
    return (x - m) / jnp.sqrt(v + eps)


def _b16(x):
    return x.astype(jnp.bfloat16)


def _b16f(x):
    return x.astype(jnp.bfloat16).astype(jnp.float32)


def _mm(w, x):
    return jnp.dot(_b16(w), _b16(x), preferred_element_type=jnp.float32)


def _k1_body(xl_ref, te_ref, xh_ref,
             qW, qb, kW, kb, vW, vb, oW, ob, f1W, f1b, f2W, f2b,
             cW0, cW1, cb,
             xl1_ref, xh1_ref):
    nb = xl_ref.shape[-1]
    tn = T * nb
    x3 = xl_ref[0] + te_ref[0]
    xm = x3.reshape(F, tn)
    q = (_mm(qW[...], xm) + qb[...]).reshape(H, D, T, nb)
    k = (_mm(kW[...], xm) + kb[...]).reshape(H, D, T, nb)
    v = (_mm(vW[...], xm) + vb[...]).reshape(H, D, T, nb)
    ti = lax.broadcasted_iota(jnp.int32, (T, T, 1), 0)
    si = lax.broadcasted_iota(jnp.int32, (T, T, 1), 1)
    causal = si <= ti
    scale = 1.0 / math.sqrt(D)
    heads = []
    for h in range(H):
        qh, kh, vh = _b16f(q[h]), _b16f(k[h]), _b16f(v[h])
        att = (qh[:, :, None, :] * kh[:, None, :, :]).sum(0) * scale
        att = jnp.where(causal, att, -32767.0)
        mx = att.max(1, keepdims=True)
        e = jnp.exp(att - mx)
        att = _b16f(e / e.sum(1, keepdims=True))
        heads.append((att[None] * vh[:, None, :, :]).sum(2))
    val = jnp.concatenate(heads, axis=0).reshape(F, tn)
    o = _mm(oW[...], val) + ob[...]
    y = _lnf(o + xm)
    h1 = jax.nn.relu(_mm(f1W[...], y) + f1b[...])
    h2 = _mm(f2W[...], h1) + f2b[...]
    xl1_ref[...] = _lnf(h2 + y).reshape(1, F, T, nb)

    xhm = xh_ref[0].reshape(F, tn)
    y1 = (_mm(cW1[...], xhm) + cb[...]).reshape(F, T, nb)
    y0 = _mm(cW0[...], xhm).reshape(F, T, nb)
    y0s = jnp.concatenate([jnp.zeros((F, 1, nb), jnp.float32), y0[:, : T - 1, :]], axis=1)
    xh1_ref[...] = jax.nn.relu(y1 + y0s).reshape(1, F, T, nb)


def _k2_body(xl1_ref, xh1_ref, adjT_ref, svecT_ref, tvecT_ref, sval, tval,
             lqW, lqb, lkW, lkb, lvW, lvb,
             hqW, hqb, hkW, hkb, hvW, hvb,
             ql_ref, kl_ref, vl_ref, qh_ref, kh_ref, vh_ref, oh_ref):
    nb = xl1_ref.shape[-1]
    tn = T * nb
    a1 = svecT_ref[...] * sval[...]
    a2 = tvecT_ref[...] * tval[...]
    ci = lax.broadcasted_iota(jnp.int32, (N, nb), 0)
    for j in range(LA):
        oh_ref[j] = (ci == adjT_ref[j, :][None, :]).astype(jnp.bfloat16)

    for x_ref, qW, qb, kW, kb, vW, vb, q_ref, k_ref, v_ref in (
        (xl1_ref, lqW, lqb, lkW, lkb, lvW, lvb, ql_ref, kl_ref, vl_ref),
        (xh1_ref, hqW, hqb, hkW, hkb, hvW, hvb, qh_ref, kh_ref, vh_ref),
    ):
        xm = ((x_ref[0] + a1[:, None, :]) + a2[:, None, :]).reshape(F, tn)
        q_ref[...] = (_mm(qW[...], xm) + qb[...]).reshape(1, F, T, nb)
        k_ref[...] = (_mm(kW[...], xm) + kb[...]).reshape(1, F, T, nb)
        v_ref[...] = (_mm(vW[...], xm) + vb[...]).reshape(1, F, T, nb)


def _sparse_branch(q, k, v, oh_ref, pW, pb):
    scale = 1.0 / math.sqrt(D)

    m = jnp.zeros((T, N), jnp.float32)
    for j in range(LA):
        ksj = jnp.dot(k.reshape(D * T, N), oh_ref[j].astype(jnp.float32),
                      preferred_element_type=jnp.float32,
                      precision=lax.Precision.HIGHEST).reshape(D, T, N)
        qks = (q * ksj).sum(0)
        m = m + _b16f(qks) * _b16f(pW[0, j])
    m = m + pb

    lane = lax.broadcasted_iota(jnp.int32, (T, N), 1)
    mm_ = m
    idxs = []
    for _ in range(NSAMP):
        mx = mm_.max(1, keepdims=True)
        idx = jnp.min(jnp.where(mm_ == mx, lane, N), 1, keepdims=True)
        idxs.append(idx)
        mm_ = jnp.where(lane == idx, -1e30, mm_)

    R = NSAMP * T
    idxb = jnp.concatenate(idxs, axis=0)
    lane_r = lax.broadcasted_iota(jnp.int32, (R, N), 1)
    ohb = (lane_r == idxb).astype(jnp.bfloat16)
    q2 = q.reshape(D * T, N)
    k2 = k.reshape(D * T, N)
    v2 = v.reshape(D * T, N)
    qT = _b16(q2.T)
    qred = jnp.dot(ohb, qT, preferred_element_type=jnp.float32)
    r_t = lax.rem(lax.broadcasted_iota(jnp.int32, (R, 1), 0), T)
    c_t = lax.rem(lax.broadcasted_iota(jnp.int32, (R, D * T), 1), T)
    qred = jnp.where(r_t == c_t, qred, 0.0)
    qk = jnp.dot(_b16(qred), _b16(k2),
                 preferred_element_type=jnp.float32) * scale
    mx = qk.max(1, keepdims=True)
    e = jnp.exp(qk - mx)
    attn = e / e.sum(1, keepdims=True)

    vft = jnp.dot(_b16(v2), _b16(attn.T),
                  preferred_element_type=jnp.float32)
    vr_t = lax.rem(lax.broadcasted_iota(jnp.int32, (D * T, 1), 0), T)
    vc_t = lax.rem(lax.broadcasted_iota(jnp.int32, (D * T, R), 1), T)
    vft = jnp.where(vr_t == vc_t, vft, 0.0)

    best = attn[0:T, :]
    bidx = jnp.zeros((T, N), jnp.int32)
    for i in range(1, NSAMP):
        ai = attn[i * T:(i + 1) * T, :]
        gt = ai > best
        best = jnp.where(gt, ai, best)
        bidx = jnp.where(gt, i, bidx)

    bidx_rep = jnp.concatenate([bidx] * NSAMP, axis=0)
    r_i = lax.div(lax.broadcasted_iota(jnp.int32, (R, N), 0), T)
    cpoh = (bidx_rep == r_i).astype(jnp.float32)
    acc = jnp.dot(vft, cpoh, preferred_element_type=jnp.float32,
                  precision=lax.Precision.HIGHEST)
    return acc.reshape(D, T, N)


def _k3_body(ql_ref, kl_ref, vl_ref, qh_ref, kh_ref, vh_ref,
             oh_ref, pWl, pbl, pWh, pbh, vall_ref, valh_ref):
    vall_ref[...] = _sparse_branch(
        ql_ref[0, 0], kl_ref[0, 0], vl_ref[0, 0], oh_ref, pWl, pbl[0, 0]
    )[None, None]
    valh_ref[...] = _sparse_branch(
        qh_ref[0, 0], kh_ref[0, 0], vh_ref[0, 0], oh_ref, pWh, pbh[0, 0]
    )[None, None]


def _k4_body(vall_ref, valh_ref, xl1_ref, xh1_ref,
             loW, lob, llng, llnb, lf1W, lf1b, lf2W, lf2b,
             hoW, hob, hlng, hlnb, hf1W, hf1b, hf2W, hf2b,
             outl_ref, outh_ref):
    nb = vall_ref.shape[-1]
    tn = T * nb
    for val_ref, x1_ref, oW, ob, lng, lnb, f1W, f1b, f2W, f2b, out_ref in (
        (vall_ref, xl1_ref, loW, lob, llng, llnb, lf1W, lf1b, lf2W, lf2b, outl_ref),
        (valh_ref, xh1_ref, hoW, hob, hlng, hlnb, hf1W, hf1b, hf2W, hf2b, outh_ref),
    ):
        vm = val_ref[0].reshape(F, tn)
        o = _mm(oW[...], vm) + ob[...]
        y = _lnf(o) * lng[...] + lnb[...]
        h1 = jax.nn.relu(_mm(f1W[...], y) + f1b[...])
        h2 = _mm(f2W[...], h1) + f2b[...]
        s = _lnf(h2 + y) + x1_ref[0].reshape(F, tn)
        out_ref[...] = s.reshape(1, F, T, nb)


def _full(shape):
    nd = len(shape)
    return pl.BlockSpec(shape, lambda *_: (0,) * nd)


def kernel(xl, xh, te, localadj, params):
    f32 = jnp.float32
    tem, spl, sph = params['tem'], params['spa_l'], params['spa_h']

    xl_f = jnp.transpose(xl, (0, 3, 1, 2))
    xh_f = jnp.transpose(xh, (0, 3, 1, 2))
    te_f = jnp.transpose(te, (0, 3, 1, 2))
    adjT = localadj.T
    svecT = params['spa_vec'].T
    tvecT = params['tem_vec'].T
    sval = params['spa_val'][:, None]
    tval = params['tem_val'][:, None]

    def col(p, names):
        out = []
        for nm in names:
            w = p[nm]
            out.append(w[:, None] if w.ndim == 1 else w)
        return out

    tem_args = col(tem, ['qW', 'qb', 'kW', 'kb', 'vW', 'vb', 'oW', 'ob',
                         'f1W', 'f1b', 'f2W', 'f2b'])
    cW0 = params['convW'][:, :, 0, 0]
    cW1 = params['convW'][:, :, 0, 1]
    cb = params['convb'][:, None]

    dblk = pl.BlockSpec((1, F, T, NB_BLK), lambda b, n: (b, 0, 0, n))
    dshape = jax.ShapeDtypeStruct((B, F, T, N), f32)

    xl1_f, xh1_f = pl.pallas_call(
        _k1_body,
        grid=(B, NBLKS),
        in_specs=[dblk, dblk, dblk] + [_full(a.shape) for a in tem_args]
                 + [_full(cW0.shape), _full(cW1.shape), _full(cb.shape)],
        out_specs=[dblk, dblk],
        out_shape=[dshape, dshape],
        interpret=_INTERPRET,
    )(xl_f, te_f, xh_f, *tem_args, cW0, cW1, cb)

    dblk2 = pl.BlockSpec((1, F, T, NB_BLK), lambda n, b: (b, 0, 0, n))
    qkv_args_l = col(spl, ['qW', 'qb', 'kW', 'kb', 'vW', 'vb'])
    qkv_args_h = col(sph, ['qW', 'qb', 'kW', 'kb', 'vW', 'vb'])
    k2_out = pl.pallas_call(
        _k2_body,
        grid=(NBLKS, B),
        in_specs=[dblk2, dblk2,
                  pl.BlockSpec((LA, NB_BLK), lambda n, b: (0, n)),
                  pl.BlockSpec((F, NB_BLK), lambda n, b: (0, n)),
                  pl.BlockSpec((F, NB_BLK), lambda n, b: (0, n)),
                  _full(sval.shape), _full(tval.shape)]
                 + [_full(a.shape) for a in qkv_args_l]
                 + [_full(a.shape) for a in qkv_args_h],
        out_specs=[dblk2] * 6
                  + [pl.BlockSpec((LA, N, NB_BLK), lambda n, b: (0, 0, n))],
        out_shape=[dshape] * 6
                  + [jax.ShapeDtypeStruct((LA, N, N), jnp.bfloat16)],
        interpret=_INTERPRET,
    )(xl1_f, xh1_f, adjT, svecT, tvecT, sval, tval, *qkv_args_l, *qkv_args_h)
    ql, kl, vl, qh, kh, vh, OH = k2_out

    h5 = lambda a: a.reshape(B, H, D, T, N)
    hblk = pl.BlockSpec((1, 1, D, T, N), lambda b, h: (b, h, 0, 0, 0))
    hshape = jax.ShapeDtypeStruct((B, H, D, T, N), f32)
    pbl = spl['projb'].reshape(1, 1)
    pbh = sph['projb'].reshape(1, 1)
    vall5, valh5 = pl.pallas_call(
        _k3_body,
        grid=(B, H),
        in_specs=[hblk] * 6 + [_full((LA, N, N)),
                               _full(spl['projW'].shape), _full((1, 1)),
                               _full(sph['projW'].shape), _full((1, 1))],
        out_specs=[hblk, hblk],
        out_shape=[hshape, hshape],
        interpret=_INTERPRET,
    )(h5(ql), h5(kl), h5(vl), h5(qh), h5(kh), h5(vh), OH,
      spl['projW'], pbl, sph['projW'], pbh)

    out_args_l = col(spl, ['oW', 'ob', 'lng', 'lnb', 'f1W', 'f1b', 'f2W', 'f2b'])
    out_args_h = col(sph, ['oW', 'ob', 'lng', 'lnb', 'f1W', 'f1b', 'f2W', 'f2b'])
    outl_f, outh_f = pl.pallas_call(
        _k4_body,
        grid=(B, NBLKS),
        in_specs=[dblk] * 4 + [_full(a.shape) for a in out_args_l]
                 + [_full(a.shape) for a in out_args_h],
        out_specs=[dblk, dblk],
        out_shape=[dshape, dshape],
        interpret=_INTERPRET,
    )(vall5.reshape(B, F, T, N), valh5.reshape(B, F, T, N), xl1_f, xh1_f,
      *out_args_l, *out_args_h)

    outl = jnp.transpose(outl_f, (0, 2, 3, 1))
    outh = jnp.transpose(outh_f, (0, 2, 3, 1))
    return (outl, outh)


# speedup vs baseline: 9.8477x; 1.1685x over previous
import functools
import math

import jax
import jax.numpy as jnp
from jax import lax
from jax.experimental import pallas as pl
from jax.experimental.pallas import tpu as pltpu
from jax.experimental.pallas import tpu_sc as plsc

B, T, N, H, D = 2, 12, 512, 8, 16
F = H * D
LA = 8
NSAMP = int(2 * math.log2(N))
NB_BLK = 128
NBLKS = N // NB_BLK
_INTERPRET = False


def _lnf(x, eps=1e-5):
    m = jnp.mean(x, 0, keepdims=True)
    v = jnp.mean((x - m) ** 2, 0, keepdims=True)
    return (x - m) / jnp.sqrt(v + eps)


def _b16(x):
    return x.astype(jnp.bfloat16)


def _b16f(x):
    return x.astype(jnp.bfloat16).astype(jnp.float32)


def _mm(w, x):
    return jnp.dot(_b16(w), _b16(x), preferred_element_type=jnp.float32)


def _k1_body(xl_ref, te_ref, xh_ref,
             qW, qb, kW, kb, vW, vb, oW, ob, f1W, f1b, f2W, f2b,
             cW0, cW1, cb,
             xl1_ref, xh1_ref):
    nb = xl_ref.shape[-1]
    tn = T * nb
    x3 = xl_ref[0] + te_ref[0]
    xm = x3.reshape(F, tn)
    q = (_mm(qW[...], xm) + qb[...]).reshape(H, D, T, nb)
    k = (_mm(kW[...], xm) + kb[...]).reshape(H, D, T, nb)
    v = (_mm(vW[...], xm) + vb[...]).reshape(H, D, T, nb)
    ti = lax.broadcasted_iota(jnp.int32, (T, T, 1), 0)
    si = lax.broadcasted_iota(jnp.int32, (T, T, 1), 1)
    causal = si <= ti
    scale = 1.0 / math.sqrt(D)
    heads = []
    for h in range(H):
        qh, kh, vh = _b16f(q[h]), _b16f(k[h]), _b16f(v[h])
        att = (qh[:, :, None, :] * kh[:, None, :, :]).sum(0) * scale
        att = jnp.where(causal, att, -32767.0)
        mx = att.max(1, keepdims=True)
        e = jnp.exp(att - mx)
        att = _b16f(e / e.sum(1, keepdims=True))
        heads.append((att[None] * vh[:, None, :, :]).sum(2))
    val = jnp.concatenate(heads, axis=0).reshape(F, tn)
    o = _mm(oW[...], val) + ob[...]
    y = _lnf(o + xm)
    h1 = jax.nn.relu(_mm(f1W[...], y) + f1b[...])
    h2 = _mm(f2W[...], h1) + f2b[...]
    xl1_ref[...] = _lnf(h2 + y).reshape(1, F, T, nb)

    xhm = xh_ref[0].reshape(F, tn)
    y1 = (_mm(cW1[...], xhm) + cb[...]).reshape(F, T, nb)
    y0 = _mm(cW0[...], xhm).reshape(F, T, nb)
    y0s = jnp.concatenate([jnp.zeros((F, 1, nb), jnp.float32), y0[:, : T - 1, :]], axis=1)
    xh1_ref[...] = jax.nn.relu(y1 + y0s).reshape(1, F, T, nb)


def _k2_body(xl1_ref, xh1_ref, svecT_ref, tvecT_ref, sval, tval,
             lqW, lqb, lkW, lkb, lvW, lvb,
             hqW, hqb, hkW, hkb, hvW, hvb,
             ql_ref, kl_ref, vl_ref, qh_ref, kh_ref, vh_ref):
    nb = xl1_ref.shape[-1]
    tn = T * nb
    a1 = svecT_ref[...] * sval[...]
    a2 = tvecT_ref[...] * tval[...]

    for x_ref, qW, qb, kW, kb, vW, vb, q_ref, k_ref, v_ref in (
        (xl1_ref, lqW, lqb, lkW, lkb, lvW, lvb, ql_ref, kl_ref, vl_ref),
        (xh1_ref, hqW, hqb, hkW, hkb, hvW, hvb, qh_ref, kh_ref, vh_ref),
    ):
        xm = ((x_ref[0] + a1[:, None, :]) + a2[:, None, :]).reshape(F, tn)
        q_ref[...] = (_mm(qW[...], xm) + qb[...]).reshape(1, F, T, nb)
        k_ref[...] = (_mm(kW[...], xm) + kb[...]).reshape(1, F, T, nb)
        v_ref[...] = (_mm(vW[...], xm) + vb[...]).reshape(1, F, T, nb)


def _sc_m_body(ql_h, kl_h, qh_h, kh_h, adj_h, pwl_h, pbl_h, pwh_h, pbh_h,
               ml_h, mh_h, adj_v, pwl_v, pbl_v, pwh_v, pbh_v, kv, qv, mv):
    info = plsc.get_sparse_core_info()
    nw = info.num_cores * info.num_subcores
    tpw = (B * H * T) // nw
    wid = lax.axis_index("s") * info.num_cores + lax.axis_index("c")
    pltpu.sync_copy(adj_h, adj_v)
    pltpu.sync_copy(pwl_h, pwl_v)
    pltpu.sync_copy(pbl_h, pbl_v)
    pltpu.sync_copy(pwh_h, pwh_v)
    pltpu.sync_copy(pbh_h, pbh_v)

    def do_branch(q_h, k_h, m_h, pw_v, pb_v):
        def task(s, carry):
            tid = wid * tpw + s
            b = tid // (H * T)
            r = tid - b * (H * T)
            h = r // T
            t = r - h * T

            pltpu.sync_copy(k_h.at[b, h, :, t, :], kv)
            pltpu.sync_copy(q_h.at[b, h, :, t, :], qv)

            def chunk(c, carry2):
                off = c * 16
                macc = jnp.zeros((16,), jnp.float32)
                for j in range(LA):
                    idx = adj_v[j, pl.ds(off, 16)]
                    prods = []
                    for d in range(D):
                        kvv = plsc.load_gather(
                            kv, [jnp.full((16,), d, jnp.int32), idx])
                        prods.append(qv[d, pl.ds(off, 16)] * kvv)
                    while len(prods) > 1:
                        prods = [prods[i] + prods[i + 1]
                                 for i in range(0, len(prods), 2)]
                    qks = prods[0]
                    bqk = plsc.bitcast(qks, jnp.uint32)
                    bqk = (bqk + jnp.uint32(0x7FFF)
                           + ((bqk >> jnp.uint32(16)) & jnp.uint32(1))) \
                        & jnp.uint32(0xFFFF0000)
                    macc = macc + plsc.bitcast(bqk, jnp.float32) * pw_v[j]
                mv[pl.ds(off, 16)] = macc + pb_v[...]
                return carry2

            lax.fori_loop(0, N // 16, chunk, 0)
            pltpu.sync_copy(mv, m_h.at[b, h, t, :])
            return carry

        lax.fori_loop(0, tpw, task, 0)

    do_branch(ql_h, kl_h, ml_h, pwl_v, pbl_v)
    do_branch(qh_h, kh_h, mh_h, pwh_v, pbh_v)


def _sc_m(ql5, kl5, qh5, kh5, adjT, pwl, pbl, pwh, pbh):
    f32 = jnp.float32
    mesh = plsc.VectorSubcoreMesh(core_axis_name="c", subcore_axis_name="s")
    fn = pl.kernel(
        _sc_m_body, mesh=mesh,
        compiler_params=pltpu.CompilerParams(needs_layout_passes=False),
        out_type=[jax.ShapeDtypeStruct((B, H, T, N), f32)] * 2,
        scratch_types=[
            pltpu.VMEM((LA, N), jnp.int32),
            pltpu.VMEM((LA, 16), f32), pltpu.VMEM((16,), f32),
            pltpu.VMEM((LA, 16), f32), pltpu.VMEM((16,), f32),
            pltpu.VMEM((D, N), f32), pltpu.VMEM((D, N), f32),
            pltpu.VMEM((N,), f32),
        ],
    )
    return fn(ql5, kl5, qh5, kh5, adjT, pwl, pbl, pwh, pbh)


def _sparse_branch(q, k, v, m):
    scale = 1.0 / math.sqrt(D)

    lane = lax.broadcasted_iota(jnp.int32, (T, N), 1)
    mm_ = m
    idxs = []
    for _ in range(NSAMP):
        mx = mm_.max(1, keepdims=True)
        idx = jnp.min(jnp.where(mm_ == mx, lane, N), 1, keepdims=True)
        idxs.append(idx)
        mm_ = jnp.where(lane == idx, -1e30, mm_)

    R = NSAMP * T
    idxb = jnp.concatenate(idxs, axis=0)
    lane_r = lax.broadcasted_iota(jnp.int32, (R, N), 1)
    ohb = (lane_r == idxb).astype(jnp.bfloat16)
    q2 = q.reshape(D * T, N)
    k2 = k.reshape(D * T, N)
    v2 = v.reshape(D * T, N)
    qT = _b16(q2.T)
    qred = jnp.dot(ohb, qT, preferred_element_type=jnp.float32)
    r_t = lax.rem(lax.broadcasted_iota(jnp.int32, (R, 1), 0), T)
    c_t = lax.rem(lax.broadcasted_iota(jnp.int32, (R, D * T), 1), T)
    qred = jnp.where(r_t == c_t, qred, 0.0)
    qk = jnp.dot(_b16(qred), _b16(k2),
                 preferred_element_type=jnp.float32) * scale
    mx = qk.max(1, keepdims=True)
    e = jnp.exp(qk - mx)
    attn = e / e.sum(1, keepdims=True)

    vft = jnp.dot(_b16(v2), _b16(attn.T),
                  preferred_element_type=jnp.float32)
    vr_t = lax.rem(lax.broadcasted_iota(jnp.int32, (D * T, 1), 0), T)
    vc_t = lax.rem(lax.broadcasted_iota(jnp.int32, (D * T, R), 1), T)
    vft = jnp.where(vr_t == vc_t, vft, 0.0)

    best = attn[0:T, :]
    bidx = jnp.zeros((T, N), jnp.int32)
    for i in range(1, NSAMP):
        ai = attn[i * T:(i + 1) * T, :]
        gt = ai > best
        best = jnp.where(gt, ai, best)
        bidx = jnp.where(gt, i, bidx)

    bidx_rep = jnp.concatenate([bidx] * NSAMP, axis=0)
    r_i = lax.div(lax.broadcasted_iota(jnp.int32, (R, N), 0), T)
    cpoh = (bidx_rep == r_i).astype(jnp.float32)
    acc = jnp.dot(vft, cpoh, preferred_element_type=jnp.float32,
                  precision=lax.Precision.HIGHEST)
    return acc.reshape(D, T, N)


def _k3_body(ql_ref, kl_ref, vl_ref, qh_ref, kh_ref, vh_ref,
             ml_ref, mh_ref, vall_ref, valh_ref):
    vall_ref[...] = _sparse_branch(
        ql_ref[0, 0], kl_ref[0, 0], vl_ref[0, 0], ml_ref[0, 0])[None, None]
    valh_ref[...] = _sparse_branch(
        qh_ref[0, 0], kh_ref[0, 0], vh_ref[0, 0], mh_ref[0, 0])[None, None]


def _k4_body(vall_ref, valh_ref, xl1_ref, xh1_ref,
             loW, lob, llng, llnb, lf1W, lf1b, lf2W, lf2b,
             hoW, hob, hlng, hlnb, hf1W, hf1b, hf2W, hf2b,
             outl_ref, outh_ref):
    nb = vall_ref.shape[-1]
    tn = T * nb
    for val_ref, x1_ref, oW, ob, lng, lnb, f1W, f1b, f2W, f2b, out_ref in (
        (vall_ref, xl1_ref, loW, lob, llng, llnb, lf1W, lf1b, lf2W, lf2b, outl_ref),
        (valh_ref, xh1_ref, hoW, hob, hlng, hlnb, hf1W, hf1b, hf2W, hf2b, outh_ref),
    ):
        vm = val_ref[0].reshape(F, tn)
        o = _mm(oW[...], vm) + ob[...]
        y = _lnf(o) * lng[...] + lnb[...]
        h1 = jax.nn.relu(_mm(f1W[...], y) + f1b[...])
        h2 = _mm(f2W[...], h1) + f2b[...]
        s = _lnf(h2 + y) + x1_ref[0].reshape(F, tn)
        out_ref[...] = s.reshape(1, F, T, nb)


def _full(shape):
    nd = len(shape)
    return pl.BlockSpec(shape, lambda *_: (0,) * nd)


def kernel(xl, xh, te, localadj, params):
    f32 = jnp.float32
    tem, spl, sph = params['tem'], params['spa_l'], params['spa_h']

    xl_f = jnp.transpose(xl, (0, 3, 1, 2))
    xh_f = jnp.transpose(xh, (0, 3, 1, 2))
    te_f = jnp.transpose(te, (0, 3, 1, 2))
    adjT = localadj.T
    svecT = params['spa_vec'].T
    tvecT = params['tem_vec'].T
    sval = params['spa_val'][:, None]
    tval = params['tem_val'][:, None]

    def col(p, names):
        out = []
        for nm in names:
            w = p[nm]
            out.append(w[:, None] if w.ndim == 1 else w)
        return out

    tem_args = col(tem, ['qW', 'qb', 'kW', 'kb', 'vW', 'vb', 'oW', 'ob',
                         'f1W', 'f1b', 'f2W', 'f2b'])
    cW0 = params['convW'][:, :, 0, 0]
    cW1 = params['convW'][:, :, 0, 1]
    cb = params['convb'][:, None]

    dblk = pl.BlockSpec((1, F, T, NB_BLK), lambda b, n: (b, 0, 0, n))
    dshape = jax.ShapeDtypeStruct((B, F, T, N), f32)

    xl1_f, xh1_f = pl.pallas_call(
        _k1_body,
        grid=(B, NBLKS),
        in_specs=[dblk, dblk, dblk] + [_full(a.shape) for a in tem_args]
                 + [_full(cW0.shape), _full(cW1.shape), _full(cb.shape)],
        out_specs=[dblk, dblk],
        out_shape=[dshape, dshape],
        interpret=_INTERPRET,
    )(xl_f, te_f, xh_f, *tem_args, cW0, cW1, cb)

    dblk2 = pl.BlockSpec((1, F, T, NB_BLK), lambda n, b: (b, 0, 0, n))
    qkv_args_l = col(spl, ['qW', 'qb', 'kW', 'kb', 'vW', 'vb'])
    qkv_args_h = col(sph, ['qW', 'qb', 'kW', 'kb', 'vW', 'vb'])
    k2_out = pl.pallas_call(
        _k2_body,
        grid=(NBLKS, B),
        in_specs=[dblk2, dblk2,
                  pl.BlockSpec((F, NB_BLK), lambda n, b: (0, n)),
                  pl.BlockSpec((F, NB_BLK), lambda n, b: (0, n)),
                  _full(sval.shape), _full(tval.shape)]
                 + [_full(a.shape) for a in qkv_args_l]
                 + [_full(a.shape) for a in qkv_args_h],
        out_specs=[dblk2] * 6,
        out_shape=[dshape] * 6,
        interpret=_INTERPRET,
    )(xl1_f, xh1_f, svecT, tvecT, sval, tval, *qkv_args_l, *qkv_args_h)
    ql, kl, vl, qh, kh, vh = k2_out

    h5 = lambda a: a.reshape(B, H, D, T, N)
    ql5, kl5, vl5 = h5(ql), h5(kl), h5(vl)
    qh5, kh5, vh5 = h5(qh), h5(kh), h5(vh)
    pwl = jnp.broadcast_to(
        _b16f(spl['projW']).reshape(LA, 1), (LA, 16))
    pwh = jnp.broadcast_to(
        _b16f(sph['projW']).reshape(LA, 1), (LA, 16))
    pbl = jnp.broadcast_to(spl['projb'].reshape(1), (16,))
    pbh = jnp.broadcast_to(sph['projb'].reshape(1), (16,))
    ml, mh = _sc_m(ql5, kl5, qh5, kh5, adjT, pwl, pbl, pwh, pbh)

    hblk = pl.BlockSpec((1, 1, D, T, N), lambda b, h: (b, h, 0, 0, 0))
    mblk = pl.BlockSpec((1, 1, T, N), lambda b, h: (b, h, 0, 0))
    hshape = jax.ShapeDtypeStruct((B, H, D, T, N), f32)
    vall5, valh5 = pl.pallas_call(
        _k3_body,
        grid=(B, H),
        in_specs=[hblk] * 6 + [mblk, mblk],
        out_specs=[hblk, hblk],
        out_shape=[hshape, hshape],
        interpret=_INTERPRET,
    )(ql5, kl5, vl5, qh5, kh5, vh5, ml, mh)

    out_args_l = col(spl, ['oW', 'ob', 'lng', 'lnb', 'f1W', 'f1b', 'f2W', 'f2b'])
    out_args_h = col(sph, ['oW', 'ob', 'lng', 'lnb', 'f1W', 'f1b', 'f2W', 'f2b'])
    outl_f, outh_f = pl.pallas_call(
        _k4_body,
        grid=(B, NBLKS),
        in_specs=[dblk] * 4 + [_full(a.shape) for a in out_args_l]
                 + [_full(a.shape) for a in out_args_h],
        out_specs=[dblk, dblk],
        out_shape=[dshape, dshape],
        interpret=_INTERPRET,
    )(vall5.reshape(B, F, T, N), valh5.reshape(B, F, T, N), xl1_f, xh1_f,
      *out_args_l, *out_args_h)

    outl = jnp.transpose(outl_f, (0, 2, 3, 1))
    outh = jnp.transpose(outh_f, (0, 2, 3, 1))
    return (outl, outh)
